# Initial kernel scaffold; baseline (speedup 1.0000x reference)
import jax
import jax.numpy as jnp
from jax.experimental import pallas as pl


def kernel(x, sorted_sequence):
    raise NotImplementedError("write your pallas kernel here")


# speedup vs baseline: 8.0636x; 8.0636x over previous
import jax
import jax.numpy as jnp
from jax.experimental import pallas as pl
from jax.experimental.pallas import tpu as pltpu


def _body(s_ref, x_ref, o_ref):
    x = x_ref[...]
    k = s_ref.shape[0]
    acc = jnp.full(x.shape, k, jnp.int32)
    for j in range(k):
        acc -= (x <= s_ref[j]).astype(jnp.int32)
    o_ref[...] = acc


def kernel(x, sorted_sequence):
    n = x.shape[0]
    cols = 1024
    rows = n // cols
    block_rows = 1024
    x2 = x.reshape(rows, cols)
    out = pl.pallas_call(
        _body,
        grid=(rows // block_rows,),
        in_specs=[
            pl.BlockSpec(memory_space=pltpu.SMEM),
            pl.BlockSpec((block_rows, cols), lambda i: (i, 0)),
        ],
        out_specs=pl.BlockSpec((block_rows, cols), lambda i: (i, 0)),
        out_shape=jax.ShapeDtypeStruct((rows, cols), jnp.int32),
    )(sorted_sequence, x2)
    return out.reshape(n)
